# baseline (speedup 1.0000x reference)
import functools

import jax
import jax.numpy as jnp
import numpy as np
from jax import lax
from jax.experimental import pallas as pl
from jax.experimental.pallas import tpu as pltpu
from jax.experimental.pallas import tpu_sc as plsc

VOCAB = 100000
D_MODEL = 128
BATCH = 4096
SEQ = 200
SCALE = float(np.sqrt(D_MODEL))

CHUNK = 40
NCHUNK = SEQ // CHUNK
UNITS = BATCH * NCHUNK
NSLOT = 2 * NCHUNK


def _positional_encoding(length, depth):
    positions = np.arange(length)[:, np.newaxis]
    depths = np.arange(depth // 2)[np.newaxis, :] / (depth // 2)
    angle_rates = 1.0 / (10000 ** depths)
    angle_rads = positions * angle_rates
    return np.concatenate(
        [np.sin(angle_rads), np.cos(angle_rads)], axis=-1
    ).astype(np.float32)


_PE = _positional_encoding(SEQ, D_MODEL)


def _make_sc_kernel():
    info = plsc.get_sparse_core_info()
    nc, ns, lanes = info.num_cores, info.num_subcores, info.num_lanes
    nw = nc * ns
    seq_per_w = BATCH // nw
    u_per_w = UNITS // nw
    iters = seq_per_w // 2
    half_u = u_per_w // 2
    half_i = iters // 2
    nvec = D_MODEL // lanes
    mesh = plsc.VectorSubcoreMesh(core_axis_name="c", subcore_axis_name="s")

    @functools.partial(
        pl.kernel,
        mesh=mesh,
        out_type=jax.ShapeDtypeStruct((BATCH, SEQ, D_MODEL), jnp.float32),
        scratch_types=[
            pltpu.VMEM((half_u, CHUNK), jnp.int32),
            pltpu.VMEM((SEQ, D_MODEL), jnp.float32),
        ]
        + [pltpu.VMEM((CHUNK, D_MODEL), jnp.float32)] * NSLOT
        + [pltpu.SemaphoreType.DMA] * (2 * NSLOT),
    )
    def k(idx_hbm, table_hbm, pe_hbm, out_hbm, idx_v, pe_v, *bufs):
        slots = bufs[:NSLOT]
        gsem = bufs[NSLOT:2 * NSLOT]
        ssem = bufs[2 * NSLOT:3 * NSLOT]
        wid = lax.axis_index("s") * nc + lax.axis_index("c")
        sbase = wid * seq_per_w
        pltpu.sync_copy(pe_hbm, pe_v)
        pltpu.sync_copy(idx_hbm.at[pl.ds(wid * u_per_w, half_u)], idx_v)
        for s in range(NSLOT):
            pltpu.async_copy(table_hbm.at[idx_v.at[s]], slots[s], gsem[s])

        def body(i, carry):
            for s in range(NSLOT):
                j, c = divmod(s, NCHUNK)
                seq = sbase + 2 * i + j
                pltpu.make_async_copy(
                    out_hbm.at[sbase, pl.ds(c * CHUNK, CHUNK)],
                    slots[s], gsem[s],
                ).wait()

                def rbody(t, c2, s=s, c=c):
                    for r in range(4):
                        jj = 4 * t + r
                        for v in range(nvec):
                            sl = pl.ds(v * lanes, lanes)
                            slots[s][jj, sl] = (
                                slots[s][jj, sl] * SCALE
                                + pe_v[c * CHUNK + jj, sl]
                            )
                    return c2

                lax.fori_loop(0, CHUNK // 4, rbody, 0)
                pltpu.async_copy(
                    slots[s],
                    out_hbm.at[seq, pl.ds(c * CHUNK, CHUNK)],
                    ssem[s],
                )

            @pl.when(i == half_i - 1)
            def _refill_idx():
                pltpu.sync_copy(
                    idx_hbm.at[pl.ds(wid * u_per_w + half_u, half_u)], idx_v
                )

            for s in range(NSLOT):
                j, c = divmod(s, NCHUNK)

                @pl.when(i < iters - 1)
                def _next(i=i, s=s, j=j, c=c):
                    pltpu.make_async_copy(
                        slots[s],
                        out_hbm.at[sbase, pl.ds(c * CHUNK, CHUNK)],
                        ssem[s],
                    ).wait()
                    u_next = NSLOT * (i + 1) + s
                    u_next = u_next - jnp.where(
                        i >= half_i - 1, NSLOT * half_i, 0
                    )
                    pltpu.async_copy(
                        table_hbm.at[idx_v.at[u_next]], slots[s], gsem[s]
                    )
            return carry

        lax.fori_loop(0, iters, body, 0)
        for s in range(NSLOT):
            j, c = divmod(s, NCHUNK)
            pltpu.make_async_copy(
                slots[s],
                out_hbm.at[sbase, pl.ds(c * CHUNK, CHUNK)],
                ssem[s],
            ).wait()

    return k


_sc_kernel = _make_sc_kernel()


def kernel(inputs, table):
    idx = inputs.reshape(UNITS, CHUNK)
    return _sc_kernel(idx, table, jnp.asarray(_PE))


# speedup vs baseline: 1.1834x; 1.1834x over previous
import functools

import jax
import jax.numpy as jnp
import numpy as np
from jax import lax
from jax.experimental import pallas as pl
from jax.experimental.pallas import tpu as pltpu
from jax.experimental.pallas import tpu_sc as plsc

VOCAB = 100000
D_MODEL = 128
BATCH = 4096
SEQ = 200
SCALE = float(np.sqrt(D_MODEL))

CHUNK = 40
NCHUNK = SEQ // CHUNK
UNITS = BATCH * NCHUNK


def _positional_encoding(length, depth):
    positions = np.arange(length)[:, np.newaxis]
    depths = np.arange(depth // 2)[np.newaxis, :] / (depth // 2)
    angle_rates = 1.0 / (10000 ** depths)
    angle_rads = positions * angle_rates
    return np.concatenate(
        [np.sin(angle_rads), np.cos(angle_rads)], axis=-1
    ).astype(np.float32)


_PE = _positional_encoding(SEQ, D_MODEL)


def _make_sc_kernel():
    info = plsc.get_sparse_core_info()
    nc, ns, lanes = info.num_cores, info.num_subcores, info.num_lanes
    nw = nc * ns
    seq_per_w = BATCH // nw
    u_per_w = UNITS // nw
    nvec = D_MODEL // lanes
    mesh = plsc.VectorSubcoreMesh(core_axis_name="c", subcore_axis_name="s")

    @functools.partial(
        pl.kernel,
        mesh=mesh,
        out_type=jax.ShapeDtypeStruct((BATCH, SEQ, D_MODEL), jnp.float32),
        scratch_types=[
            pltpu.VMEM((u_per_w // 2, CHUNK), jnp.int32),
            pltpu.VMEM((SEQ, D_MODEL), jnp.float32),
        ]
        + [pltpu.VMEM((CHUNK, D_MODEL), jnp.float32)] * (2 * NCHUNK)
        + [pltpu.SemaphoreType.DMA] * (2 * NCHUNK + 1),
    )
    def k(idx_hbm, table_hbm, pe_hbm, out_hbm, idx_v, pe_v, *bufs):
        rows_in = bufs[:NCHUNK]
        rows_out = bufs[NCHUNK:2 * NCHUNK]
        gsem = bufs[2 * NCHUNK:3 * NCHUNK]
        ssem = bufs[3 * NCHUNK:4 * NCHUNK]
        psem = bufs[4 * NCHUNK]
        wid = lax.axis_index("s") * nc + lax.axis_index("c")
        sbase = wid * seq_per_w
        half_u = u_per_w // 2
        half_i = seq_per_w // 2
        pltpu.sync_copy(idx_hbm.at[pl.ds(wid * u_per_w, half_u)], idx_v)
        pltpu.async_copy(pe_hbm, pe_v, psem)
        for c in range(NCHUNK):
            pltpu.async_copy(table_hbm.at[idx_v.at[c]], rows_in[c], gsem[c])
        pltpu.make_async_copy(pe_hbm, pe_v, psem).wait()

        def body(i, carry):
            seq = sbase + i

            @pl.when(i == half_i - 1)
            def _refill_idx():
                for c in range(NCHUNK):
                    pltpu.make_async_copy(
                        out_hbm.at[sbase, pl.ds(c * CHUNK, CHUNK)],
                        rows_in[c], gsem[c],
                    ).wait()
                pltpu.sync_copy(
                    idx_hbm.at[pl.ds(wid * u_per_w + half_u, half_u)], idx_v
                )

            for c in range(NCHUNK):
                @pl.when(i != half_i - 1)
                def _wait_gather(c=c):
                    pltpu.make_async_copy(
                        out_hbm.at[sbase, pl.ds(c * CHUNK, CHUNK)],
                        rows_in[c], gsem[c],
                    ).wait()

                @pl.when(i > 0)
                def _wait_store(c=c):
                    pltpu.make_async_copy(
                        rows_out[c],
                        out_hbm.at[sbase, pl.ds(c * CHUNK, CHUNK)],
                        ssem[c],
                    ).wait()

                def rbody(j, c2, c=c):
                    for r in range(4):
                        jj = 4 * j + r
                        for v in range(nvec):
                            sl = pl.ds(v * lanes, lanes)
                            rows_out[c][jj, sl] = (
                                rows_in[c][jj, sl] * SCALE
                                + pe_v[c * CHUNK + jj, sl]
                            )
                    return c2

                lax.fori_loop(0, CHUNK // 4, rbody, 0)
                pltpu.async_copy(
                    rows_out[c],
                    out_hbm.at[seq, pl.ds(c * CHUNK, CHUNK)],
                    ssem[c],
                )

                @pl.when(i < seq_per_w - 1)
                def _fire_next(i=i, c=c):
                    u_next = NCHUNK * (i + 1) + c
                    u_next = u_next - jnp.where(
                        i >= half_i - 1, NCHUNK * half_i, 0
                    )
                    pltpu.async_copy(
                        table_hbm.at[idx_v.at[u_next]],
                        rows_in[c],
                        gsem[c],
                    )
            return carry

        lax.fori_loop(0, seq_per_w, body, 0)
        for c in range(NCHUNK):
            pltpu.make_async_copy(
                rows_out[c],
                out_hbm.at[sbase, pl.ds(c * CHUNK, CHUNK)],
                ssem[c],
            ).wait()

    return k


_sc_kernel = _make_sc_kernel()


def kernel(inputs, table):
    idx = inputs.reshape(UNITS, CHUNK)
    return _sc_kernel(idx, table, jnp.asarray(_PE))


# speedup vs baseline: 1.1857x; 1.0019x over previous
import functools

import jax
import jax.numpy as jnp
import numpy as np
from jax import lax
from jax.experimental import pallas as pl
from jax.experimental.pallas import tpu as pltpu
from jax.experimental.pallas import tpu_sc as plsc

VOCAB = 100000
D_MODEL = 128
BATCH = 4096
SEQ = 200
SCALE = float(np.sqrt(D_MODEL))

CHUNK = 40
NCHUNK = SEQ // CHUNK
UNITS = BATCH * NCHUNK


def _positional_encoding(length, depth):
    positions = np.arange(length)[:, np.newaxis]
    depths = np.arange(depth // 2)[np.newaxis, :] / (depth // 2)
    angle_rates = 1.0 / (10000 ** depths)
    angle_rads = positions * angle_rates
    return np.concatenate(
        [np.sin(angle_rads), np.cos(angle_rads)], axis=-1
    ).astype(np.float32)


_PE = _positional_encoding(SEQ, D_MODEL)


def _make_sc_kernel():
    info = plsc.get_sparse_core_info()
    nc, ns, lanes = info.num_cores, info.num_subcores, info.num_lanes
    nw = nc * ns
    seq_per_w = BATCH // nw
    u_per_w = UNITS // nw
    nvec = D_MODEL // lanes
    mesh = plsc.VectorSubcoreMesh(core_axis_name="c", subcore_axis_name="s")

    @functools.partial(
        pl.kernel,
        mesh=mesh,
        out_type=jax.ShapeDtypeStruct((BATCH, SEQ, D_MODEL), jnp.float32),
        scratch_types=[
            pltpu.VMEM((u_per_w // 2, CHUNK), jnp.int32),
            pltpu.VMEM((SEQ, D_MODEL), jnp.float32),
        ]
        + [pltpu.VMEM((CHUNK, D_MODEL), jnp.float32)] * (2 * NCHUNK)
        + [pltpu.SemaphoreType.DMA] * (2 * NCHUNK + 1),
    )
    def k(idx_hbm, table_hbm, pe_hbm, out_hbm, idx_v, pe_v, *bufs):
        rows_in = bufs[:NCHUNK]
        rows_out = bufs[NCHUNK:2 * NCHUNK]
        gsem = bufs[2 * NCHUNK:3 * NCHUNK]
        ssem = bufs[3 * NCHUNK:4 * NCHUNK]
        psem = bufs[4 * NCHUNK]
        wid = lax.axis_index("s") * nc + lax.axis_index("c")
        sbase = wid * seq_per_w
        half_u = u_per_w // 2
        half_i = seq_per_w // 2
        pltpu.sync_copy(idx_hbm.at[pl.ds(wid * u_per_w, half_u)], idx_v)
        pltpu.async_copy(pe_hbm, pe_v, psem)
        for c in range(NCHUNK):
            pltpu.async_copy(table_hbm.at[idx_v.at[c]], rows_in[c], gsem[c])
        pltpu.make_async_copy(pe_hbm, pe_v, psem).wait()

        def body(i, carry):
            seq = sbase + i

            @pl.when(i == half_i - 1)
            def _refill_idx():
                for c in range(NCHUNK):
                    pltpu.make_async_copy(
                        out_hbm.at[sbase, pl.ds(c * CHUNK, CHUNK)],
                        rows_in[c], gsem[c],
                    ).wait()
                pltpu.sync_copy(
                    idx_hbm.at[pl.ds(wid * u_per_w + half_u, half_u)], idx_v
                )

            for c in range(NCHUNK):
                @pl.when(i != half_i - 1)
                def _wait_gather(c=c):
                    pltpu.make_async_copy(
                        out_hbm.at[sbase, pl.ds(c * CHUNK, CHUNK)],
                        rows_in[c], gsem[c],
                    ).wait()

                @pl.when(i > 0)
                def _wait_store(c=c):
                    pltpu.make_async_copy(
                        rows_out[c],
                        out_hbm.at[sbase, pl.ds(c * CHUNK, CHUNK)],
                        ssem[c],
                    ).wait()

                def rbody(j, c2, c=c):
                    for r in range(4):
                        jj = 4 * j + r
                        for v in range(nvec):
                            sl = pl.ds(v * lanes, lanes)
                            rows_out[c][jj, sl] = (
                                rows_in[c][jj, sl] * SCALE
                                + pe_v[c * CHUNK + jj, sl]
                            )
                    return c2

                lax.fori_loop(0, CHUNK // 4, rbody, 0)

                @pl.when(i < seq_per_w - 1)
                def _fire_next(i=i, c=c):
                    u_next = NCHUNK * (i + 1) + c
                    u_next = u_next - jnp.where(
                        i >= half_i - 1, NCHUNK * half_i, 0
                    )
                    pltpu.async_copy(
                        table_hbm.at[idx_v.at[u_next]],
                        rows_in[c],
                        gsem[c],
                    )

                pltpu.async_copy(
                    rows_out[c],
                    out_hbm.at[seq, pl.ds(c * CHUNK, CHUNK)],
                    ssem[c],
                )
            return carry

        lax.fori_loop(0, seq_per_w, body, 0)
        for c in range(NCHUNK):
            pltpu.make_async_copy(
                rows_out[c],
                out_hbm.at[sbase, pl.ds(c * CHUNK, CHUNK)],
                ssem[c],
            ).wait()

    return k


_sc_kernel = _make_sc_kernel()


def kernel(inputs, table):
    idx = inputs.reshape(UNITS, CHUNK)
    return _sc_kernel(idx, table, jnp.asarray(_PE))
